# baseline (speedup 1.0000x reference)
import functools

import jax
import jax.numpy as jnp
from jax import lax
from jax.experimental import pallas as pl
from jax.experimental.pallas import tpu as pltpu
from jax.experimental.pallas import tpu_sc as plsc

BATCH = 16384
EMBED_DIM = 32
PACK = 4
L = 16


@functools.lru_cache(maxsize=None)
def _build_gather():
    info = plsc.get_sparse_core_info()
    nw = info.num_cores * info.num_subcores
    bpw = BATCH // nw
    opw = bpw * EMBED_DIM
    nblk = bpw // L
    mesh = plsc.VectorSubcoreMesh(core_axis_name="c", subcore_axis_name="s")

    @functools.partial(
        pl.kernel,
        out_type=jax.ShapeDtypeStruct((BATCH * EMBED_DIM,), jnp.float32),
        mesh=mesh,
        compiler_params=pltpu.CompilerParams(needs_layout_passes=False),
        scratch_types=[
            pltpu.VMEM((bpw,), jnp.int32),
            pltpu.VMEM((bpw,), jnp.int32),
            pltpu.VMEM((bpw, PACK * EMBED_DIM), jnp.float32),
            pltpu.VMEM((opw,), jnp.float32),
            pltpu.SemaphoreType.DMA,
        ],
    )
    def gather(idx_hbm, table_hbm, out_hbm, idx_v, q_v, rows_v, out_v, sem):
        wid = lax.axis_index("s") * info.num_cores + lax.axis_index("c")
        pltpu.sync_copy(idx_hbm.at[pl.ds(wid * bpw, bpw)], idx_v)

        def qblock(b, carry):
            v = idx_v[pl.ds(b * L, L)]
            q_v[pl.ds(b * L, L)] = lax.shift_right_logical(v, 2)
            return carry

        lax.fori_loop(0, nblk, qblock, 0)
        pltpu.async_copy(table_hbm.at[q_v], rows_v, sem).wait()

        iota = lax.iota(jnp.int32, L)

        def eblock(b, carry):
            i0 = b * L
            v = idx_v[pl.ds(i0, L)]
            roff = (v & 3) * EMBED_DIM
            srcrow = i0 + iota
            dstbase = srcrow * EMBED_DIM
            for cc in range(EMBED_DIM):
                x = plsc.load_gather(rows_v, [srcrow, roff + cc])
                plsc.store_scatter(out_v, [dstbase + cc], x)
            return carry

        lax.fori_loop(0, nblk, eblock, 0)
        pltpu.sync_copy(out_v, out_hbm.at[pl.ds(wid * opw, opw)])

    return gather


def kernel(user_inputs, userEmbedding):
    idx = user_inputs.astype(jnp.int32)
    table = userEmbedding.reshape(userEmbedding.shape[0] // PACK, PACK * EMBED_DIM)
    flat = _build_gather()(idx, table)
    return flat.reshape(BATCH, EMBED_DIM)
